# baseline (speedup 1.0000x reference)
import functools

import numpy as np
import jax
import jax.numpy as jnp
from jax import lax
from jax.experimental import pallas as pl
from jax.experimental.pallas import tpu as pltpu
from jax.experimental.pallas import tpu_sc as plsc

_FIELDS = 26
_FIELD_DIM = 100000
_EMBED = 32
_BATCH = 16384
_N = _BATCH * _FIELDS
_NC, _NS, _L = 2, 16, 16
_NW = _NC * _NS
_NPW = _N // _NW
_CHUNK = 128
_NCHUNK = _NPW // _CHUNK
_K = 8
_NGRP = _NCHUNK // _K
_NSET = 3

_PATTERN = np.asarray(
    (np.arange(_NPW, dtype=np.int64) % _FIELDS) * _FIELD_DIM, dtype=np.int32
)

_ROWS = _FIELDS * _FIELD_DIM

_TBW = 8192
_TGRID = -(-_ROWS // _TBW)


def _tp_body(tin, tout):
    tout[...] = tin[...].T


_transpose_table = pl.pallas_call(
    _tp_body,
    grid=(_TGRID,),
    in_specs=[pl.BlockSpec((_EMBED, _TBW), lambda i: (0, i))],
    out_specs=pl.BlockSpec((_TBW, _EMBED), lambda i: (i, 0)),
    out_shape=jax.ShapeDtypeStruct((_ROWS, _EMBED), jnp.float32),
)

_mesh = plsc.VectorSubcoreMesh(core_axis_name="c", subcore_axis_name="s")


@functools.partial(
    pl.kernel,
    mesh=_mesh,
    out_type=jax.ShapeDtypeStruct((_N, _EMBED), jnp.float32),
    scratch_types=[
        pltpu.VMEM((_NPW,), jnp.int32),
        pltpu.VMEM((_NPW,), jnp.int32),
        pltpu.VMEM((_NSET, _K, _CHUNK, _EMBED), jnp.float32),
        pltpu.SemaphoreType.DMA,
        pltpu.SemaphoreType.DMA,
    ],
    compiler_params=pltpu.CompilerParams(use_tc_tiling_on_sc=False),
)
def _embedding_gather(x_hbm, pat_hbm, table_hbm, out_hbm,
                      idx_v, pat_v, bufs, gsem, wsem):
    wid = lax.axis_index("s") * _NC + lax.axis_index("c")
    base = wid * _NPW

    pltpu.sync_copy(x_hbm.at[pl.ds(base, _NPW)], idx_v)
    pltpu.sync_copy(pat_hbm, pat_v)

    grp_vecs = _K * _CHUNK // _L

    def _add_group(grp):
        def _add(i, carry):
            s = grp * _K * _CHUNK + i * _L
            idx_v[pl.ds(s, _L)] = idx_v[pl.ds(s, _L)] + pat_v[pl.ds(s, _L)]
            return carry

        lax.fori_loop(0, grp_vecs, _add, 0)

    _add_group(0)

    write_q = []
    for grp in range(_NGRP):
        par = grp % _NSET
        gathers = []
        for j in range(_K):
            g = grp * _K + j
            gathers.append(
                pltpu.async_copy(
                    table_hbm.at[idx_v.at[pl.ds(g * _CHUNK, _CHUNK)]],
                    bufs.at[par, j],
                    gsem,
                )
            )
        if grp + 1 < _NGRP:
            _add_group(grp + 1)
        if len(write_q) >= _NSET - 1:
            for w in write_q.pop(0):
                w.wait()
        for cp in gathers:
            cp.wait()
        writes = []
        for j in range(_K):
            g = grp * _K + j
            writes.append(
                pltpu.async_copy(
                    bufs.at[par, j],
                    out_hbm.at[pl.ds(base + g * _CHUNK, _CHUNK)],
                    wsem,
                )
            )
        write_q.append(writes)
    for ws in write_q:
        for w in ws:
            w.wait()


def kernel(x, table):
    table_lin = _transpose_table(table.T)
    xflat = x.reshape(_N)
    pat = jnp.asarray(_PATTERN)
    out = _embedding_gather(xflat, pat, table_lin)
    return out.reshape(_BATCH, _FIELDS, _EMBED)


# speedup vs baseline: 1.7566x; 1.7566x over previous
import functools

import numpy as np
import jax
import jax.numpy as jnp
from jax import lax
from jax.experimental import pallas as pl
from jax.experimental.pallas import tpu as pltpu
from jax.experimental.pallas import tpu_sc as plsc

_FIELDS = 26
_FIELD_DIM = 100000
_EMBED = 32
_BATCH = 16384
_N = _BATCH * _FIELDS
_NC, _NS, _L = 2, 16, 16
_NW = _NC * _NS
_BPW = _BATCH // _NW
_NPW = _BPW * _FIELDS
_CHUNK = 128
_TPW = _BPW // _CHUNK
_CGRP = _EMBED // 8

_ROWS = _FIELDS * _FIELD_DIM

_PACK = 128 // _EMBED
_TBW = 2048
_TGRID = -(-_ROWS // (_PACK * _TBW))
_PROWS_PAD = _TGRID * _TBW
_LASTBLK = _ROWS // _TBW


def _tp_body(t0, t1, t2, t3, tout):
    tout[...] = jnp.concatenate(
        [t0[...].T, t1[...].T, t2[...].T, t3[...].T], axis=1)


_transpose_table = pl.pallas_call(
    _tp_body,
    grid=(_TGRID,),
    in_specs=[
        pl.BlockSpec(
            (_EMBED, _TBW),
            lambda i, q=q: (0, jnp.minimum(_PACK * i + q, _LASTBLK)))
        for q in range(_PACK)
    ],
    out_specs=pl.BlockSpec((_TBW, 128), lambda i: (i, 0)),
    out_shape=jax.ShapeDtypeStruct((_PROWS_PAD, 128), jnp.float32),
)

_mesh = plsc.VectorSubcoreMesh(core_axis_name="c", subcore_axis_name="s")


@functools.partial(
    pl.kernel,
    mesh=_mesh,
    out_type=jax.ShapeDtypeStruct((_FIELDS, _CGRP, _BATCH // _CHUNK, 8, _CHUNK),
                                  jnp.float32),
    scratch_types=[
        pltpu.VMEM((_NPW,), jnp.int32),
        pltpu.VMEM((2, _TPW, _CHUNK), jnp.int32),
        pltpu.VMEM((2, _TPW, _CHUNK, _EMBED), jnp.float32),
        pltpu.VMEM((2, _TPW, _CGRP, 8, _CHUNK), jnp.float32),
        pltpu.SemaphoreType.DMA,
        pltpu.SemaphoreType.DMA,
    ],
    compiler_params=pltpu.CompilerParams(use_tc_tiling_on_sc=False,
                                         needs_layout_passes=False),
)
def _embedding_gather(x_hbm, table_hbm, out_hbm,
                      xv, gidx, bufs, tbufs, gsem, wsem):
    wid = lax.axis_index("s") * _NC + lax.axis_index("c")
    base = wid * _NPW
    pltpu.sync_copy(x_hbm.at[pl.ds(base, _NPW)], xv)

    iota = lax.iota(jnp.int32, _L)

    def _build_indices(par, f):
        off = f * _FIELD_DIM
        for t in range(_TPW):
            for k in range(_CHUNK // _L):
                xpos = iota * _FIELDS + ((t * _CHUNK + k * _L) * _FIELDS + f)
                r = plsc.load_gather(xv, [xpos]) + off
                gidx[par, t, pl.ds(k * _L, _L)] = (
                    ((r >> 13) << 13) + ((r & 2047) << 2) + ((r >> 11) & 3))

    def _fire_gathers(par):
        return [
            pltpu.async_copy(table_hbm.at[gidx.at[par, t]],
                             bufs.at[par, t], gsem)
            for t in range(_TPW)
        ]

    def _wait_one_gather(par, t):
        pltpu.make_async_copy(table_hbm.at[gidx.at[par, t]],
                              bufs.at[par, t], gsem).wait()

    def _wait_one_write(par, t):
        pltpu.make_async_copy(tbufs.at[par, t],
                              out_hbm.at[0, :, 0, :, :], wsem).wait()

    def _transpose_chunk(par, t):
        def _col(c, carry):
            cg = c // 8
            s = c % 8
            for k in range(_CHUNK // _L):
                rows = iota + (k * _L)
                vals = plsc.load_gather(bufs.at[par, t],
                                        [rows, jnp.broadcast_to(c, (_L,))])
                tbufs[par, t, cg, s, pl.ds(k * _L, _L)] = vals
            return carry

        lax.fori_loop(0, _EMBED, _col, 0)

    def _fire_write(par, t, f):
        return pltpu.async_copy(
            tbufs.at[par, t],
            out_hbm.at[f, :, wid * _TPW + t, :, :],
            wsem,
        )

    def _head(par, f):
        _build_indices(par, f)
        _fire_gathers(par)

    _head(0, 0)

    def _steady(f, carry):
        par = f % 2

        def _one_phase(par):
            opar = 1 - par
            _build_indices(par, f)
            @pl.when(f >= 2)
            def _():
                for t in range(_TPW):
                    _wait_one_write(par, t)
            _fire_gathers(par)
            for t in range(_TPW):
                _wait_one_gather(opar, t)
                _transpose_chunk(opar, t)
                _fire_write(opar, t, f - 1)

        @pl.when(par == 0)
        def _():
            _one_phase(0)

        @pl.when(par == 1)
        def _():
            _one_phase(1)

        return carry

    lax.fori_loop(1, _FIELDS, _steady, 0)

    lpar = (_FIELDS - 1) % 2
    for t in range(_TPW):
        _wait_one_gather(lpar, t)
        _transpose_chunk(lpar, t)
        _fire_write(lpar, t, _FIELDS - 1)
    for t in range(_TPW):
        _wait_one_write(1 - lpar, t)
        _wait_one_write(lpar, t)


def kernel(x, table):
    tt = table.T
    table_lin = _transpose_table(tt, tt, tt, tt).reshape(
        _PROWS_PAD * _PACK, _EMBED)
    xflat = x.reshape(_N)
    out5 = _embedding_gather(xflat, table_lin)
    return out5.transpose(2, 4, 0, 1, 3).reshape(_BATCH, _FIELDS, _EMBED)


# speedup vs baseline: 2.3703x; 1.3494x over previous
import functools

import numpy as np
import jax
import jax.numpy as jnp
from jax import lax
from jax.experimental import pallas as pl
from jax.experimental.pallas import tpu as pltpu
from jax.experimental.pallas import tpu_sc as plsc

_FIELDS = 26
_FIELD_DIM = 100000
_EMBED = 32
_BATCH = 16384
_N = _BATCH * _FIELDS
_NC, _NS, _L = 2, 16, 16
_NW = _NC * _NS
_BPW = _BATCH // _NW
_NPW = _BPW * _FIELDS
_CHUNK = 128
_TPW = _BPW // _CHUNK
_CGRP = _EMBED // 8

_ROWS = _FIELDS * _FIELD_DIM

_PACK = 128 // _EMBED
_TBW = 4096
_TGRID = -(-_ROWS // (_PACK * _TBW))
_PROWS_PAD = _TGRID * _TBW
_LASTBLK = _ROWS // _TBW


def _tp_body(t0, t1, t2, t3, tout):
    dn = (((0,), (0,)), ((), ()))
    rows = lax.broadcasted_iota(jnp.int32, (_EMBED, 128), 0)
    cols = lax.broadcasted_iota(jnp.int32, (_EMBED, 128), 1)
    acc = None
    for q, tq in enumerate((t0, t1, t2, t3)):
        eq = (cols == rows + 32 * q).astype(jnp.float32)
        d = lax.dot_general(tq[...], eq, dn,
                            preferred_element_type=jnp.float32)
        acc = d if acc is None else acc + d
    tout[...] = acc


_transpose_table = pl.pallas_call(
    _tp_body,
    grid=(_TGRID,),
    in_specs=[
        pl.BlockSpec(
            (_EMBED, _TBW),
            lambda i, q=q: (0, jnp.minimum(_PACK * i + q, _LASTBLK)))
        for q in range(_PACK)
    ],
    out_specs=pl.BlockSpec((_TBW, 128), lambda i: (i, 0)),
    out_shape=jax.ShapeDtypeStruct((_PROWS_PAD, 128), jnp.float32),
)

_mesh = plsc.VectorSubcoreMesh(core_axis_name="c", subcore_axis_name="s")


@functools.partial(
    pl.kernel,
    mesh=_mesh,
    out_type=jax.ShapeDtypeStruct((_FIELDS, _CGRP, _BATCH // _CHUNK, 8, _CHUNK),
                                  jnp.float32),
    scratch_types=[
        pltpu.VMEM((_NPW,), jnp.int32),
        pltpu.VMEM((2, _TPW, _CHUNK), jnp.int32),
        pltpu.VMEM((2, _TPW, _CHUNK, _EMBED), jnp.float32),
        pltpu.VMEM((2, _TPW, _CGRP, 8, _CHUNK), jnp.float32),
        pltpu.SemaphoreType.DMA,
        pltpu.SemaphoreType.DMA,
    ],
    compiler_params=pltpu.CompilerParams(use_tc_tiling_on_sc=False,
                                         needs_layout_passes=False),
)
def _embedding_gather(x_hbm, table_hbm, out_hbm,
                      xv, gidx, bufs, tbufs, gsem, wsem):
    wid = lax.axis_index("s") * _NC + lax.axis_index("c")
    base = wid * _NPW
    pltpu.sync_copy(x_hbm.at[pl.ds(base, _NPW)], xv)

    iota = lax.iota(jnp.int32, _L)

    def _build_indices(par, f):
        off = f * _FIELD_DIM
        for t in range(_TPW):
            for k in range(_CHUNK // _L):
                xpos = iota * _FIELDS + ((t * _CHUNK + k * _L) * _FIELDS + f)
                r = plsc.load_gather(xv, [xpos]) + off
                gidx[par, t, pl.ds(k * _L, _L)] = (
                    ((r >> 14) << 14) + ((r & 4095) << 2) + ((r >> 12) & 3))

    def _fire_gathers(par):
        return [
            pltpu.async_copy(table_hbm.at[gidx.at[par, t]],
                             bufs.at[par, t], gsem)
            for t in range(_TPW)
        ]

    def _wait_one_gather(par, t):
        pltpu.make_async_copy(table_hbm.at[gidx.at[par, t]],
                              bufs.at[par, t], gsem).wait()

    def _wait_one_write(par, t):
        pltpu.make_async_copy(tbufs.at[par, t],
                              out_hbm.at[0, :, 0, :, :], wsem).wait()

    def _transpose_chunk(par, t):
        def _col(c, carry):
            cg = c // 8
            s = c % 8
            for k in range(_CHUNK // _L):
                rows = iota + (k * _L)
                vals = plsc.load_gather(bufs.at[par, t],
                                        [rows, jnp.broadcast_to(c, (_L,))])
                tbufs[par, t, cg, s, pl.ds(k * _L, _L)] = vals
            return carry

        lax.fori_loop(0, _EMBED, _col, 0)

    def _fire_write(par, t, f):
        return pltpu.async_copy(
            tbufs.at[par, t],
            out_hbm.at[f, :, wid * _TPW + t, :, :],
            wsem,
        )

    def _head(par, f):
        _build_indices(par, f)
        _fire_gathers(par)

    _head(0, 0)

    def _steady(f, carry):
        par = f % 2

        def _one_phase(par):
            opar = 1 - par
            _build_indices(par, f)
            @pl.when(f >= 2)
            def _():
                for t in range(_TPW):
                    _wait_one_write(par, t)
            _fire_gathers(par)
            for t in range(_TPW):
                _wait_one_gather(opar, t)
                _transpose_chunk(opar, t)
                _fire_write(opar, t, f - 1)

        @pl.when(par == 0)
        def _():
            _one_phase(0)

        @pl.when(par == 1)
        def _():
            _one_phase(1)

        return carry

    lax.fori_loop(1, _FIELDS, _steady, 0)

    lpar = (_FIELDS - 1) % 2
    for t in range(_TPW):
        _wait_one_gather(lpar, t)
        _transpose_chunk(lpar, t)
        _fire_write(lpar, t, _FIELDS - 1)
    for t in range(_TPW):
        _wait_one_write(1 - lpar, t)
        _wait_one_write(lpar, t)


def kernel(x, table):
    tt = table.T
    table_lin = _transpose_table(tt, tt, tt, tt).reshape(
        _PROWS_PAD * _PACK, _EMBED)
    xflat = x.reshape(_N)
    out5 = _embedding_gather(xflat, table_lin)
    return out5.transpose(2, 4, 0, 1, 3).reshape(_BATCH, _FIELDS, _EMBED)
